# baseline (speedup 1.0000x reference)
import jax
import jax.numpy as jnp
from jax.experimental import pallas as pl
from jax.experimental.pallas import tpu as pltpu


def _proj_body(x_ref, w_ref, b_ref, xp_ref):
    bs = x_ref.shape[0]
    f = w_ref.shape[0]
    tb = x_ref.shape[1] // f
    w = w_ref[...]
    b = b_ref[...]
    for k in range(tb):
        xb = x_ref[:, k * f:(k + 1) * f].astype(jnp.bfloat16)
        xp_ref[k * bs:(k + 1) * bs, :] = (
            jnp.dot(xb, w, preferred_element_type=jnp.float32) + b)


def _cell_body(xp_ref, a_ref, whq_ref, bq_ref, wd_ref, bd_ref, out_ref,
               h_ref, c_ref):
    h_dim = h_ref.shape[1]

    @pl.when(pl.program_id(0) == 0)
    def _():
        h_ref[...] = jnp.zeros_like(h_ref)
        c_ref[...] = jnp.zeros_like(c_ref)

    h = h_ref[...]
    c = c_ref[...]
    z = jnp.dot(h.astype(jnp.bfloat16), whq_ref[...],
                preferred_element_type=jnp.float32)
    z4 = z[:, :4 * h_dim] + xp_ref[...]
    i_t = jax.nn.sigmoid(z4[:, :h_dim])
    f_t = jax.nn.sigmoid(z4[:, h_dim:2 * h_dim])
    g_t = jnp.tanh(z4[:, 2 * h_dim:3 * h_dim])
    o_t = jax.nn.sigmoid(z4[:, 3 * h_dim:])
    q_t = jnp.tanh(z[:, 4 * h_dim:] + bq_ref[...])
    aq = jnp.dot(a_ref[...], q_t.astype(jnp.bfloat16),
                 preferred_element_type=jnp.float32)
    c_new = f_t * (c + aq) + i_t * g_t
    h_new = o_t * jnp.tanh(c_new)
    h_ref[...] = h_new
    c_ref[...] = c_new
    out_ref[...] = (jnp.dot(h_new, wd_ref[...],
                            preferred_element_type=jnp.float32) + bd_ref[...])


def kernel(x, A, w_ih, w_hh, bias, w_q, b_q, w_d, b_d):
    bs, seq, f = x.shape
    h_dim = w_hh.shape[0]
    g = 4 * h_dim

    x2 = x.reshape(bs, seq * f)
    w_ih_bf = w_ih.astype(jnp.bfloat16)
    whq = jnp.concatenate([w_hh, w_q], axis=1).astype(jnp.bfloat16)
    a_bf = A.astype(jnp.bfloat16)
    bias2 = bias.reshape(1, g)
    bq2 = b_q.reshape(1, h_dim)
    bd2 = b_d.reshape(1, 1)

    tb = 2 if seq % 2 == 0 else 1
    nblk = seq // tb
    xp = pl.pallas_call(
        _proj_body,
        out_shape=jax.ShapeDtypeStruct((seq * bs, g), jnp.float32),
        grid=(nblk,),
        in_specs=[
            pl.BlockSpec((bs, tb * f), lambda i: (0, i)),
            pl.BlockSpec((f, g), lambda i: (0, 0)),
            pl.BlockSpec((1, g), lambda i: (0, 0)),
        ],
        out_specs=pl.BlockSpec((tb * bs, g), lambda i: (i, 0)),
        compiler_params=pltpu.CompilerParams(
            dimension_semantics=("parallel",)),
    )(x2, w_ih_bf, bias2)

    out = pl.pallas_call(
        _cell_body,
        out_shape=jax.ShapeDtypeStruct((seq * bs, 1), jnp.float32),
        grid=(seq,),
        in_specs=[
            pl.BlockSpec((bs, g), lambda t: (t, 0)),
            pl.BlockSpec((bs, bs), lambda t: (0, 0)),
            pl.BlockSpec((h_dim, 5 * h_dim), lambda t: (0, 0)),
            pl.BlockSpec((1, h_dim), lambda t: (0, 0)),
            pl.BlockSpec((h_dim, 1), lambda t: (0, 0)),
            pl.BlockSpec((1, 1), lambda t: (0, 0)),
        ],
        out_specs=pl.BlockSpec((bs, 1), lambda t: (t, 0)),
        scratch_shapes=[pltpu.VMEM((bs, h_dim), jnp.float32),
                        pltpu.VMEM((bs, h_dim), jnp.float32)],
        compiler_params=pltpu.CompilerParams(
            dimension_semantics=("arbitrary",)),
    )(xp, a_bf, whq, bq2, w_d, bd2)

    return jnp.transpose(out.reshape(seq, bs), (1, 0))[:, :, None]


# speedup vs baseline: 2.0969x; 2.0969x over previous
import jax
import jax.numpy as jnp
from jax import lax
from jax.experimental import pallas as pl
from jax.experimental.pallas import tpu as pltpu


def _proj_body(x_ref, w_ref, b_ref, xp_ref):
    bb, seq, f = x_ref.shape
    g = w_ref.shape[1]
    x2 = x_ref[...].reshape(bb * seq, f).astype(jnp.bfloat16)
    w = w_ref[...].astype(jnp.bfloat16)
    z = jnp.dot(x2, w, preferred_element_type=jnp.float32) + b_ref[...]
    xp_ref[...] = jnp.transpose(z.reshape(bb, seq, g), (1, 0, 2))


def _cell_body(xp_ref, a_ref, whh_ref, wq_ref, bq_ref, wd_ref, bd_ref,
               out_ref, h_ref, c_ref, y_ref, abf_ref, whq_ref):
    t = pl.program_id(0)
    seq = y_ref.shape[0]
    hd = h_ref.shape[1]

    @pl.when(t == 0)
    def _init():
        h_ref[...] = jnp.zeros_like(h_ref)
        c_ref[...] = jnp.zeros_like(c_ref)
        abf_ref[...] = a_ref[...].astype(jnp.bfloat16)
        whq_ref[:, :4 * hd] = whh_ref[...].astype(jnp.bfloat16)
        whq_ref[:, 4 * hd:] = wq_ref[...].astype(jnp.bfloat16)

    @pl.when(t > 0)
    def _head():
        yrow = lax.dot_general(wd_ref[...], h_ref[...],
                               (((1,), (1,)), ((), ())),
                               preferred_element_type=jnp.float32)
        y_ref[pl.ds(t - 1, 1), :] = yrow

    @pl.when(t < seq)
    def _step():
        h = h_ref[...]
        c = c_ref[...]
        z = jnp.dot(h.astype(jnp.bfloat16), whq_ref[...],
                    preferred_element_type=jnp.float32)
        z4 = z[:, :4 * hd] + xp_ref[0]
        i_t = jax.nn.sigmoid(z4[:, :hd])
        f_t = jax.nn.sigmoid(z4[:, hd:2 * hd])
        g_t = jnp.tanh(z4[:, 2 * hd:3 * hd])
        o_t = jax.nn.sigmoid(z4[:, 3 * hd:])
        q_t = jnp.tanh(z[:, 4 * hd:] + bq_ref[...])
        aq = jnp.dot(abf_ref[...], q_t.astype(jnp.bfloat16),
                     preferred_element_type=jnp.float32)
        c_new = f_t * (c + aq) + i_t * g_t
        h_new = o_t * jnp.tanh(c_new)
        h_ref[...] = h_new
        c_ref[...] = c_new

    @pl.when(t == seq)
    def _emit():
        eye = (lax.broadcasted_iota(jnp.int32, (seq, seq), 0) ==
               lax.broadcasted_iota(jnp.int32, (seq, seq), 1)
               ).astype(jnp.float32)
        out_ref[...] = lax.dot_general(
            y_ref[...], eye, (((0,), (0,)), ((), ())),
            preferred_element_type=jnp.float32) + bd_ref[...]


def kernel(x, A, w_ih, w_hh, bias, w_q, b_q, w_d, b_d):
    bs, seq, f = x.shape
    hd = w_hh.shape[0]
    g = 4 * hd

    bias2 = bias.reshape(1, g)
    bq2 = b_q.reshape(1, hd)
    wd2 = w_d.reshape(1, hd)
    bd2 = b_d.reshape(1, 1)

    bb = min(64, bs)
    nblk = bs // bb
    xp = pl.pallas_call(
        _proj_body,
        out_shape=jax.ShapeDtypeStruct((seq, bs, g), jnp.float32),
        grid=(nblk,),
        in_specs=[
            pl.BlockSpec((bb, seq, f), lambda j: (j, 0, 0)),
            pl.BlockSpec((f, g), lambda j: (0, 0)),
            pl.BlockSpec((1, g), lambda j: (0, 0)),
        ],
        out_specs=pl.BlockSpec((seq, bb, g), lambda j: (0, j, 0)),
        compiler_params=pltpu.CompilerParams(
            dimension_semantics=("parallel",)),
    )(x, w_ih, bias2)

    out = pl.pallas_call(
        _cell_body,
        out_shape=jax.ShapeDtypeStruct((bs, seq), jnp.float32),
        grid=(seq + 1,),
        in_specs=[
            pl.BlockSpec((1, bs, g), lambda t: (jnp.minimum(t, seq - 1), 0, 0)),
            pl.BlockSpec((bs, bs), lambda t: (0, 0)),
            pl.BlockSpec((hd, g), lambda t: (0, 0)),
            pl.BlockSpec((hd, hd), lambda t: (0, 0)),
            pl.BlockSpec((1, hd), lambda t: (0, 0)),
            pl.BlockSpec((1, hd), lambda t: (0, 0)),
            pl.BlockSpec((1, 1), lambda t: (0, 0)),
        ],
        out_specs=pl.BlockSpec((bs, seq), lambda t: (0, 0)),
        scratch_shapes=[pltpu.VMEM((bs, hd), jnp.float32),
                        pltpu.VMEM((bs, hd), jnp.float32),
                        pltpu.VMEM((seq, bs), jnp.float32),
                        pltpu.VMEM((bs, bs), jnp.bfloat16),
                        pltpu.VMEM((hd, 5 * hd), jnp.bfloat16)],
        compiler_params=pltpu.CompilerParams(
            dimension_semantics=("arbitrary",)),
    )(xp, A, w_hh, w_q, bq2, wd2, bd2)

    return out[:, :, None]


# speedup vs baseline: 3.1213x; 1.4886x over previous
import jax
import jax.numpy as jnp
from jax import lax
from jax.experimental import pallas as pl
from jax.experimental.pallas import tpu as pltpu


def _fused_body(x_ref, wih_ref, bias_ref, a_ref, whh_ref, wq_ref, bq_ref,
                wd_ref, bd_ref, out_ref, xp_ref, y_ref):
    j = pl.program_id(0)
    nblk = pl.num_programs(0)
    bb, seq, f = x_ref.shape
    bs = a_ref.shape[0]
    hd = whh_ref.shape[0]
    g = 4 * hd

    x2 = x_ref[...].reshape(bb * seq, f).astype(jnp.bfloat16)
    w = wih_ref[...].astype(jnp.bfloat16)
    z = jnp.dot(x2, w, preferred_element_type=jnp.float32) + bias_ref[...]
    xp_ref[:, pl.ds(j * bb, bb), :] = jnp.transpose(
        z.reshape(bb, seq, g), (1, 0, 2))

    @pl.when(j == nblk - 1)
    def _recur():
        abf = a_ref[...].astype(jnp.bfloat16)
        whq = jnp.concatenate(
            [whh_ref[...], wq_ref[...]], axis=1).astype(jnp.bfloat16)
        bq = bq_ref[...]
        wd = wd_ref[...]
        h = None
        c = None
        for t in range(seq):
            if t == 0:
                z4 = xp_ref[0]
                q_t = jnp.broadcast_to(jnp.tanh(bq), (bs, hd))
            else:
                zr = jnp.dot(h.astype(jnp.bfloat16), whq,
                             preferred_element_type=jnp.float32)
                z4 = zr[:, :g] + xp_ref[t]
                q_t = jnp.tanh(zr[:, g:] + bq)
            i_t = jax.nn.sigmoid(z4[:, :hd])
            f_t = jax.nn.sigmoid(z4[:, hd:2 * hd])
            g_t = jnp.tanh(z4[:, 2 * hd:3 * hd])
            o_t = jax.nn.sigmoid(z4[:, 3 * hd:])
            aq = jnp.dot(abf, q_t.astype(jnp.bfloat16),
                         preferred_element_type=jnp.float32)
            ca = c + aq if t > 0 else aq
            c = f_t * ca + i_t * g_t
            h = o_t * jnp.tanh(c)
            y_ref[t:t + 1, :] = lax.dot_general(
                wd, h, (((1,), (1,)), ((), ())),
                preferred_element_type=jnp.float32)
        eye = (lax.broadcasted_iota(jnp.int32, (seq, seq), 0) ==
               lax.broadcasted_iota(jnp.int32, (seq, seq), 1)
               ).astype(jnp.float32)
        out_ref[...] = lax.dot_general(
            y_ref[...], eye, (((0,), (0,)), ((), ())),
            preferred_element_type=jnp.float32) + bd_ref[...]


def kernel(x, A, w_ih, w_hh, bias, w_q, b_q, w_d, b_d):
    bs, seq, f = x.shape
    hd = w_hh.shape[0]
    g = 4 * hd

    bias2 = bias.reshape(1, g)
    bq2 = b_q.reshape(1, hd)
    wd2 = w_d.reshape(1, hd)
    bd2 = b_d.reshape(1, 1)

    bb = min(64, bs)
    nblk = bs // bb
    out = pl.pallas_call(
        _fused_body,
        out_shape=jax.ShapeDtypeStruct((bs, seq), jnp.float32),
        grid=(nblk,),
        in_specs=[
            pl.BlockSpec((bb, seq, f), lambda j: (j, 0, 0)),
            pl.BlockSpec((f, g), lambda j: (0, 0)),
            pl.BlockSpec((1, g), lambda j: (0, 0)),
            pl.BlockSpec((bs, bs), lambda j: (0, 0)),
            pl.BlockSpec((hd, g), lambda j: (0, 0)),
            pl.BlockSpec((hd, hd), lambda j: (0, 0)),
            pl.BlockSpec((1, hd), lambda j: (0, 0)),
            pl.BlockSpec((1, hd), lambda j: (0, 0)),
            pl.BlockSpec((1, 1), lambda j: (0, 0)),
        ],
        out_specs=pl.BlockSpec((bs, seq), lambda j: (0, 0)),
        scratch_shapes=[pltpu.VMEM((seq, bs, g), jnp.float32),
                        pltpu.VMEM((seq, bs), jnp.float32)],
        compiler_params=pltpu.CompilerParams(
            dimension_semantics=("arbitrary",)),
    )(x, w_ih, bias2, A, w_hh, w_q, bq2, wd2, bd2)

    return out[:, :, None]


# speedup vs baseline: 3.1315x; 1.0033x over previous
import jax
import jax.numpy as jnp
from jax import lax
from jax.experimental import pallas as pl
from jax.experimental.pallas import tpu as pltpu


def _fused_body(x_ref, wih_ref, bias_ref, a_ref, whh_ref, wq_ref, bq_ref,
                wd_ref, bd_ref, out_ref, xp_ref, y_ref):
    j = pl.program_id(0)
    nblk = pl.num_programs(0)
    bb, seq, f = x_ref.shape
    bs = a_ref.shape[0]
    hd = whh_ref.shape[0]
    g = 4 * hd

    x2 = x_ref[...].reshape(bb * seq, f).astype(jnp.bfloat16)
    w = wih_ref[...].astype(jnp.bfloat16)
    z = jnp.dot(x2, w, preferred_element_type=jnp.float32) + bias_ref[...]
    xp_ref[:, pl.ds(j * bb, bb), :] = jnp.transpose(
        z.reshape(bb, seq, g), (1, 0, 2))

    @pl.when(j == nblk - 1)
    def _recur():
        abf = a_ref[...].astype(jnp.bfloat16)
        whq = jnp.concatenate(
            [whh_ref[...], wq_ref[...]], axis=1).astype(jnp.bfloat16)
        bq = bq_ref[...]
        wd = wd_ref[...]
        h = None
        c = None
        for t in range(seq):
            if t == 0:
                z4 = xp_ref[0]
                q_t = jnp.broadcast_to(jnp.tanh(bq), (bs, hd))
            else:
                zr = jnp.dot(h.astype(jnp.bfloat16), whq,
                             preferred_element_type=jnp.float32)
                z4 = zr[:, :g] + xp_ref[t]
                q_t = jnp.tanh(zr[:, g:] + bq)
            if_t = 0.5 * jnp.tanh(0.5 * z4[:, :2 * hd]) + 0.5
            i_t = if_t[:, :hd]
            f_t = if_t[:, hd:]
            g_t = jnp.tanh(z4[:, 2 * hd:3 * hd])
            o_t = 0.5 * jnp.tanh(0.5 * z4[:, 3 * hd:]) + 0.5
            aq = jnp.dot(abf, q_t.astype(jnp.bfloat16),
                         preferred_element_type=jnp.float32)
            ca = c + aq if t > 0 else aq
            c = f_t * ca + i_t * g_t
            h = o_t * jnp.tanh(c)
            y_ref[t:t + 1, :] = lax.dot_general(
                wd, h, (((1,), (1,)), ((), ())),
                preferred_element_type=jnp.float32)
        eye = (lax.broadcasted_iota(jnp.int32, (seq, seq), 0) ==
               lax.broadcasted_iota(jnp.int32, (seq, seq), 1)
               ).astype(jnp.float32)
        out_ref[...] = lax.dot_general(
            y_ref[...], eye, (((0,), (0,)), ((), ())),
            preferred_element_type=jnp.float32) + bd_ref[...]


def kernel(x, A, w_ih, w_hh, bias, w_q, b_q, w_d, b_d):
    bs, seq, f = x.shape
    hd = w_hh.shape[0]
    g = 4 * hd

    bias2 = bias.reshape(1, g)
    bq2 = b_q.reshape(1, hd)
    wd2 = w_d.reshape(1, hd)
    bd2 = b_d.reshape(1, 1)

    bb = min(64, bs)
    nblk = bs // bb
    out = pl.pallas_call(
        _fused_body,
        out_shape=jax.ShapeDtypeStruct((bs, seq), jnp.float32),
        grid=(nblk,),
        in_specs=[
            pl.BlockSpec((bb, seq, f), lambda j: (j, 0, 0)),
            pl.BlockSpec((f, g), lambda j: (0, 0)),
            pl.BlockSpec((1, g), lambda j: (0, 0)),
            pl.BlockSpec((bs, bs), lambda j: (0, 0)),
            pl.BlockSpec((hd, g), lambda j: (0, 0)),
            pl.BlockSpec((hd, hd), lambda j: (0, 0)),
            pl.BlockSpec((1, hd), lambda j: (0, 0)),
            pl.BlockSpec((1, hd), lambda j: (0, 0)),
            pl.BlockSpec((1, 1), lambda j: (0, 0)),
        ],
        out_specs=pl.BlockSpec((bs, seq), lambda j: (0, 0)),
        scratch_shapes=[pltpu.VMEM((seq, bs, g), jnp.float32),
                        pltpu.VMEM((seq, bs), jnp.float32)],
        compiler_params=pltpu.CompilerParams(
            dimension_semantics=("arbitrary",)),
    )(x, w_ih, bias2, A, w_hh, w_q, bq2, wd2, bd2)

    return out[:, :, None]
